# baseline (speedup 1.0000x reference)
import functools

import jax
import jax.numpy as jnp
from jax.experimental import pallas as pl
from jax.experimental.pallas import tpu as pltpu

_B, _S, _D, _F, _E, _K = 1, 2048, 768, 1536, 8, 2
_T = 256
_MAXB = _S // _T
_P = _K * _S
_PADTOT = _P + _E * (_T - 1) // _T * _T + _E * _T
_NSLOT = 6144


def _k1_router(res_ref, rw_ref, dst_ref, wpair_ref, nblk_ref, bbase_ref):
    x = res_ref[0]
    logits = jnp.dot(x.astype(jnp.bfloat16), rw_ref[...].T.astype(jnp.bfloat16),
                     preferred_element_type=jnp.float32)
    m = jnp.max(logits, axis=-1, keepdims=True)
    ex = jnp.exp(logits - m)
    probs = ex / jnp.sum(ex, axis=-1, keepdims=True)
    eidx = jax.lax.broadcasted_iota(jnp.int32, probs.shape, 1)
    p1 = jnp.max(probs, axis=-1, keepdims=True)
    i1 = jnp.min(jnp.where(probs >= p1, eidx, _E), axis=-1, keepdims=True)
    probs2 = jnp.where(eidx == i1, -1.0, probs)
    p2 = jnp.max(probs2, axis=-1, keepdims=True)
    i2 = jnp.min(jnp.where(probs2 >= p2, eidx, _E), axis=-1, keepdims=True)
    denom = p1 + p2 + 1e-8
    wpair_ref[0:_S, :] = p1 / denom
    wpair_ref[_S:_P, :] = p2 / denom

    lane_e = jax.lax.broadcasted_iota(jnp.int32, (_S, _E), 1)
    oh1 = (lane_e == i1).astype(jnp.float32)
    oh2 = (lane_e == i2).astype(jnp.float32)
    counts = (jnp.sum(oh1, axis=0, keepdims=True)
              + jnp.sum(oh2, axis=0, keepdims=True))
    bc = jnp.maximum(1.0, jnp.ceil(counts / _T))
    ei = jax.lax.broadcasted_iota(jnp.int32, (_E, _E), 0)
    ej = jax.lax.broadcasted_iota(jnp.int32, (_E, _E), 1)
    triu_strict = (ei < ej).astype(jnp.float32)
    bbase = jnp.dot(bc, triu_strict, preferred_element_type=jnp.float32,
                    precision=jax.lax.Precision.HIGHEST)
    opad = bbase * _T
    nblk_ref[...] = bc.astype(jnp.int32)
    bbase_ref[...] = bbase.astype(jnp.int32)

    ci = jax.lax.broadcasted_iota(jnp.int32, (512, 512), 0)
    cj = jax.lax.broadcasted_iota(jnp.int32, (512, 512), 1)
    lt_strict = (cj < ci).astype(jnp.float32)
    carry = jnp.zeros((1, _E), jnp.float32)
    for c in range(_P // 512):
        base = c * 512
        if c < _S // 512:
            ohc = (jax.lax.broadcasted_iota(jnp.int32, (512, _E), 1)
                   == i1[base:base + 512]).astype(jnp.float32)
        else:
            sbase = base - _S
            ohc = (jax.lax.broadcasted_iota(jnp.int32, (512, _E), 1)
                   == i2[sbase:sbase + 512]).astype(jnp.float32)
        excl = jnp.dot(lt_strict, ohc, preferred_element_type=jnp.float32,
                       precision=jax.lax.Precision.HIGHEST) + carry
        carry = carry + jnp.sum(ohc, axis=0, keepdims=True)
        dstc = jnp.sum(ohc * (opad + excl), axis=1, keepdims=True)
        dst_ref[base:base + 512, :] = dstc.astype(jnp.int32)


def _k3_mlp(nblk_ref, bbase_ref, x_ref, ws_ref, win_ref, bin_ref, wout_ref,
            bout_ref, y_ref):
    e = pl.program_id(0)
    b = pl.program_id(1)

    @pl.when(b < nblk_ref[e])
    def _():
        xbf = x_ref[...].astype(jnp.bfloat16)
        h = jnp.dot(xbf, win_ref[0].astype(jnp.bfloat16),
                    preferred_element_type=jnp.float32)
        h = h + bin_ref[0]
        g = 0.5 * h * (1.0 + jax.lax.erf(h * 0.7071067811865476))
        o = jnp.dot(g.astype(jnp.bfloat16), wout_ref[0].astype(jnp.bfloat16),
                    preferred_element_type=jnp.float32)
        o = o + bout_ref[0]
        y_ref[...] = o * ws_ref[...]


def _row_idx(e, b, nblk, bbase):
    return (bbase[e] + jnp.minimum(b, nblk[e] - 1), 0)


def kernel(residual, router_w, W_in, b_in, W_out, b_out):
    S, D, F, E, T = _S, _D, _F, _E, _T
    x2d = residual.reshape(S, D)
    dst, wpair, nblk, bbase = pl.pallas_call(
        _k1_router,
        grid=(1,),
        in_specs=[
            pl.BlockSpec((1, S, D), lambda i: (0, 0, 0)),
            pl.BlockSpec((E, D), lambda i: (0, 0)),
        ],
        out_specs=[
            pl.BlockSpec((_P, 1), lambda i: (0, 0)),
            pl.BlockSpec((_P, 1), lambda i: (0, 0)),
            pl.BlockSpec((1, E), lambda i: (0, 0)),
            pl.BlockSpec((1, E), lambda i: (0, 0)),
        ],
        out_shape=[
            jax.ShapeDtypeStruct((_P, 1), jnp.int32),
            jax.ShapeDtypeStruct((_P, 1), jnp.float32),
            jax.ShapeDtypeStruct((1, E), jnp.int32),
            jax.ShapeDtypeStruct((1, E), jnp.int32),
        ],
    )(residual, router_w)

    dst1 = dst.reshape(_P)
    wp1 = wpair.reshape(_P)
    nblk1 = nblk.reshape(E)
    bbase1 = bbase.reshape(E)

    tok = jnp.concatenate([jnp.arange(S, dtype=jnp.int32)] * 2)
    src_sorted = jnp.zeros((_NSLOT,), jnp.int32).at[dst1].set(tok)
    w_slot = jnp.zeros((_NSLOT,), jnp.float32).at[dst1].set(wp1)
    x_sorted = x2d[src_sorted]

    grid_spec = pltpu.PrefetchScalarGridSpec(
        num_scalar_prefetch=2,
        grid=(E, _MAXB),
        in_specs=[
            pl.BlockSpec((T, D), _row_idx),
            pl.BlockSpec((T, 1), _row_idx),
            pl.BlockSpec((1, D, F), lambda e, b, nb, bb: (e, 0, 0)),
            pl.BlockSpec((1, 1, F), lambda e, b, nb, bb: (e, 0, 0)),
            pl.BlockSpec((1, F, D), lambda e, b, nb, bb: (e, 0, 0)),
            pl.BlockSpec((1, 1, D), lambda e, b, nb, bb: (e, 0, 0)),
        ],
        out_specs=pl.BlockSpec((T, D), _row_idx),
    )
    y = pl.pallas_call(
        _k3_mlp,
        grid_spec=grid_spec,
        out_shape=jax.ShapeDtypeStruct((_NSLOT, D), jnp.float32),
    )(nblk1, bbase1, x_sorted, w_slot.reshape(_NSLOT, 1), W_in,
      b_in.reshape(E, 1, F), W_out, b_out.reshape(E, 1, D))

    out = y[dst1[:S]] + y[dst1[S:]]
    return out.reshape(_B, S, D)
